# baseline (speedup 1.0000x reference)
import dataclasses

import jax
import jax.numpy as jnp
from jax import lax
from jax.experimental import pallas as pl
from jax.experimental.pallas import tpu as pltpu
from jax.experimental.pallas import tpu_sc as plsc

BATCH = 16384
RANK = 32
NC = 2
NS = 16
LANES = 16
NW = NC * NS
BPW = BATCH // NW
CHUNK = 128
NCH = BPW // CHUNK


def _dot_gather_body(uidx_hbm, vidx_hbm, u_hbm, v_hbm, out_hbm,
                     idx_u, idx_v, rows_u, rows_v, out_v, sem):
    wid = lax.axis_index("s") * NC + lax.axis_index("c")
    ibase = wid * NCH

    pltpu.sync_copy(uidx_hbm.at[pl.ds(ibase, NCH)], idx_u)
    pltpu.sync_copy(vidx_hbm.at[pl.ds(ibase, NCH)], idx_v)

    copies = []
    for j in range(NCH):
        copies.append(pltpu.async_copy(
            u_hbm.at[idx_u.at[j]], rows_u.at[pl.ds(j * CHUNK, CHUNK)], sem))
        copies.append(pltpu.async_copy(
            v_hbm.at[idx_v.at[j]], rows_v.at[pl.ds(j * CHUNK, CHUNK)], sem))
    for c in copies:
        c.wait()

    lane_iota = lax.iota(jnp.int32, LANES)

    @pl.loop(0, BPW // LANES)
    def _(g):
        rows16 = g * LANES + lane_iota
        acc = jnp.zeros((LANES,), jnp.float32)
        for d in range(RANK):
            dcol = jnp.full((LANES,), d, jnp.int32)
            uu = plsc.load_gather(rows_u, [rows16, dcol])
            vv = plsc.load_gather(rows_v, [rows16, dcol])
            acc = acc + uu * vv
        out_v[pl.ds(g * LANES, LANES)] = acc

    pltpu.sync_copy(out_v, out_hbm.at[pl.ds(wid * BPW, BPW)])


def kernel(idxs, U, V):
    idxs = idxs.astype(jnp.int32)
    uidx = idxs[:, 0].reshape(NW * NCH, CHUNK)
    vidx = idxs[:, 1].reshape(NW * NCH, CHUNK)
    mesh = plsc.VectorSubcoreMesh(core_axis_name="c", subcore_axis_name="s")
    cp = pltpu.CompilerParams()
    if "needs_layout_passes" in pltpu.CompilerParams.__dataclass_fields__:
        cp = dataclasses.replace(cp, needs_layout_passes=False)
    if "use_tc_tiling_on_sc" in pltpu.CompilerParams.__dataclass_fields__:
        cp = dataclasses.replace(cp, use_tc_tiling_on_sc=False)
    run = pl.kernel(
        _dot_gather_body,
        out_type=jax.ShapeDtypeStruct((BATCH,), jnp.float32),
        mesh=mesh,
        scratch_types=[
            pltpu.VMEM((NCH, CHUNK), jnp.int32),
            pltpu.VMEM((NCH, CHUNK), jnp.int32),
            pltpu.VMEM((BPW, RANK), jnp.float32),
            pltpu.VMEM((BPW, RANK), jnp.float32),
            pltpu.VMEM((BPW,), jnp.float32),
            pltpu.SemaphoreType.DMA,
        ],
        compiler_params=cp,
    )
    return run(uidx, vidx, U, V)
